# Initial kernel scaffold; baseline (speedup 1.0000x reference)
import jax
import jax.numpy as jnp
from jax.experimental import pallas as pl


def kernel(entity, adj_entity, adj_relation):
    raise NotImplementedError("write your pallas kernel here")


# speedup vs baseline: 1.0628x; 1.0628x over previous
import functools

import jax
import jax.numpy as jnp
from jax import lax
from jax.experimental import pallas as pl
from jax.experimental.pallas import tpu as pltpu
from jax.experimental.pallas import tpu_sc as plsc

_K = 8
_B = 4096
_NC = 2
_NS = 16
_NW = _NC * _NS
_BPW = _B // _NW
_H2 = _BPW * _K
_CH = 128
_NCHUNK = _H2 // _CH

_MESH = plsc.VectorSubcoreMesh(core_axis_name="c", subcore_axis_name="s")
_PARAMS = pltpu.CompilerParams(use_tc_tiling_on_sc=False)


@functools.partial(
    pl.kernel,
    mesh=_MESH,
    compiler_params=_PARAMS,
    out_type=[
        jax.ShapeDtypeStruct((_B, _K), jnp.int32),
        jax.ShapeDtypeStruct((_B, _K), jnp.int32),
    ],
    scratch_types=[
        pltpu.VMEM((_BPW,), jnp.int32),
        pltpu.VMEM((_BPW, _K), jnp.int32),
        pltpu.VMEM((_BPW, _K), jnp.int32),
        pltpu.SemaphoreType.DMA,
        pltpu.SemaphoreType.DMA,
    ],
)
def _hop1(ent_hbm, adj_e_hbm, adj_r_hbm, ent1_hbm, rel1_hbm,
          idx_v, e_v, r_v, sem_a, sem_b):
    wid = lax.axis_index("s") * _NC + lax.axis_index("c")
    base = pl.multiple_of(wid * _BPW, 8)
    pltpu.sync_copy(ent_hbm.at[pl.ds(base, _BPW)], idx_v)
    c_e = pltpu.async_copy(adj_e_hbm.at[idx_v], e_v, sem_a)
    c_r = pltpu.async_copy(adj_r_hbm.at[idx_v], r_v, sem_b)
    c_e.wait()
    w_e = pltpu.async_copy(e_v, ent1_hbm.at[pl.ds(base, _BPW)], sem_a)
    c_r.wait()
    w_r = pltpu.async_copy(r_v, rel1_hbm.at[pl.ds(base, _BPW)], sem_b)
    w_e.wait()
    w_r.wait()


@functools.partial(
    pl.kernel,
    mesh=_MESH,
    compiler_params=_PARAMS,
    out_type=[
        jax.ShapeDtypeStruct((_B * _K, _K), jnp.int32),
        jax.ShapeDtypeStruct((_B * _K, _K), jnp.int32),
    ],
    scratch_types=[
        pltpu.VMEM((_H2,), jnp.int32),
        pltpu.VMEM((_H2, _K), jnp.int32),
        pltpu.VMEM((_H2, _K), jnp.int32),
        pltpu.SemaphoreType.DMA,
        pltpu.SemaphoreType.DMA,
    ],
)
def _hop2(ent1f_hbm, adj_e_hbm, adj_r_hbm, ent2_hbm, rel2_hbm,
          idx_v, e_v, r_v, sem_a, sem_b):
    wid = lax.axis_index("s") * _NC + lax.axis_index("c")
    base = pl.multiple_of(wid * _H2, 8)
    pltpu.sync_copy(ent1f_hbm.at[pl.ds(base, _H2)], idx_v)
    copies = []
    for c in range(_NCHUNK):
        off = c * _CH
        ic = idx_v.at[pl.ds(off, _CH)]
        copies.append(pltpu.async_copy(
            adj_e_hbm.at[ic], e_v.at[pl.ds(off, _CH)], sem_a))
        copies.append(pltpu.async_copy(
            adj_r_hbm.at[ic], r_v.at[pl.ds(off, _CH)], sem_b))
    for cp in copies:
        cp.wait()
    w_e = pltpu.async_copy(e_v, ent2_hbm.at[pl.ds(base, _H2)], sem_a)
    w_r = pltpu.async_copy(r_v, rel2_hbm.at[pl.ds(base, _H2)], sem_b)
    w_e.wait()
    w_r.wait()


def kernel(entity, adj_entity, adj_relation):
    ent1, rel1 = _hop1(entity.reshape(-1), adj_entity, adj_relation)
    ent2, rel2 = _hop2(ent1.reshape(-1), adj_entity, adj_relation)
    return (
        entity,
        ent1,
        ent2.reshape(_B, _K * _K),
        rel1,
        rel2.reshape(_B, _K * _K),
    )
